# baseline (speedup 1.0000x reference)
import jax, jax.numpy as jnp
import numpy as np
from jax.experimental import pallas as pl

_AMOUNT_CLASSES = 13
_SA_CONFS = [
    {"n_balls": 512, "radius": 0.2, "K": 32, "local": [6, 64, 64], "global": [64, 128]},
    {"n_balls": 128, "radius": 0.4, "K": 64, "local": [131, 128, 128], "global": [128, 256]},
    {"n_balls": 1, "radius": np.inf, "K": 128, "local": [259, 256, 512], "global": [512, 1024]},
]
_FP_CONFS = [
    {"channels": [1027, 256, 256], "k": 1},
    {"channels": [259, 128], "k": 3},
    {"channels": [131, 128, 128, 128, _AMOUNT_CLASSES], "k": 3},
]


def _mlp(params, x, final_relu=True):
    n = len(params)
    for i, (W, b) in enumerate(params):
        x = x @ W + b
        if final_relu or i < n - 1:
            x = jax.nn.relu(x)
    return x


def _fps(pos, n):
    p = jax.lax.stop_gradient(pos)
    d = jnp.sum((p - p[0]) ** 2, axis=-1)
    idxs = jnp.zeros((n,), dtype=jnp.int32)
    def body(i, carry):
        idxs, d = carry
        nxt = jnp.argmax(d).astype(jnp.int32)
        idxs = idxs.at[i].set(nxt)
        d = jnp.minimum(d, jnp.sum((p - p[nxt]) ** 2, axis=-1))
        return (idxs, d)
    idxs, _ = jax.lax.fori_loop(1, n, body, (idxs, d))
    return idxs


def _ball_query(centers, pos, radius, K):
    c = jax.lax.stop_gradient(centers)
    p = jax.lax.stop_gradient(pos)
    d2 = jnp.sum((c[:, None, :] - p[None, :, :]) ** 2, axis=-1)
    neg, idx = jax.lax.top_k(-d2, K)
    inball = (-neg) <= radius * radius
    idx = jnp.where(inball, idx, idx[:, :1])
    return idx


def _set_abstraction(x, pos, conf, p):
    cidx = _fps(pos, conf["n_balls"])
    centers = pos[cidx]
    nidx = _ball_query(centers, pos, conf["radius"], conf["K"])
    gx = x[nidx]
    gpos = pos[nidx] - centers[:, None, :]
    h = jnp.concatenate([gx, gpos], axis=-1)
    h = _mlp(p["local"], h, True)
    h = jnp.max(h, axis=1)
    h = _mlp(p["global"], h, True)
    return h, centers


def _knn_interpolate(tpos, spos, sx, k):
    d2 = jnp.sum((tpos[:, None, :] - spos[None, :, :]) ** 2, axis=-1)
    kk = min(k, spos.shape[0])
    neg, idx = jax.lax.top_k(-d2, kk)
    w = 1.0 / (-neg + 1e-8)
    w = w / jnp.sum(w, axis=-1, keepdims=True)
    return jnp.sum(w[..., None] * sx[idx], axis=1)


def _forward_single(x, pos, params):
    old = []
    for i, conf in enumerate(_SA_CONFS):
        old.append(pos)
        x, pos = _set_abstraction(x, pos, conf, params["sa"][i])
    n_fp = len(_FP_CONFS)
    for i, conf in enumerate(_FP_CONFS):
        tpos = old[-(i + 1)]
        interp = _knn_interpolate(tpos, pos, x, conf["k"])
        h = jnp.concatenate([interp, tpos], axis=-1)
        x = _mlp(params["fp"][i], h, final_relu=(i < n_fp - 1))
        pos = tpos
    return x


def _identity_kernel(x_ref, o_ref):
    o_ref[...] = x_ref[...]


def kernel(vertex_features, vertices, params):
    out = jax.vmap(lambda x, p: _forward_single(x, p, params))(vertex_features, vertices)
    return pl.pallas_call(
        _identity_kernel,
        grid=(out.shape[0],),
        in_specs=[pl.BlockSpec((1,) + out.shape[1:], lambda i: (i, 0, 0))],
        out_specs=pl.BlockSpec((1,) + out.shape[1:], lambda i: (i, 0, 0)),
        out_shape=jax.ShapeDtypeStruct(out.shape, out.dtype),
    )(out)


# speedup vs baseline: 5.0920x; 5.0920x over previous
import jax
import jax.numpy as jnp
from jax.experimental import pallas as pl
from jax.experimental.pallas import tpu as pltpu

_B = 16
_N0 = 4096
_N1 = 512
_N2 = 128
_K1 = 32
_K2 = 64
_R1SQ = 0.2 * 0.2
_R2SQ = 0.4 * 0.4
_BIGF = 1e30
_BIGI = 2 ** 30


def _relu(x):
    return jnp.maximum(x, 0.0)


def _dot(a, b):
    return jax.lax.dot_general(a, b, (((1,), (0,)), ((), ())),
                               preferred_element_type=jnp.float32)


def _row_argmin_onehot(d2, lane):
    v = jnp.min(d2, axis=1, keepdims=True)
    li = jnp.min(jnp.where(d2 == v, lane, _BIGI), axis=1, keepdims=True)
    sel = lane == li
    return v, sel


def _pairwise_d2(c, pT):
    d0 = (c[:, 0:1] - pT[0:1, :]) ** 2
    d1 = (c[:, 1:2] - pT[1:2, :]) ** 2
    d2 = (c[:, 2:3] - pT[2:3, :]) ** 2
    return (d0 + d1) + d2


def _fps(posT, n_pts, n_centers, d_s, c_s, ct_s):
    lane = jax.lax.broadcasted_iota(jnp.int32, (1, n_pts), 1)
    row_c = jax.lax.broadcasted_iota(jnp.int32, (n_centers, 1), 0)
    lane_c = jax.lax.broadcasted_iota(jnp.int32, (1, n_centers), 1)
    p0 = posT[:, 0:1]
    p0r = jnp.concatenate([p0[0:1, :], p0[1:2, :], p0[2:3, :]], axis=1)
    d_s[...] = jnp.sum((posT - p0) ** 2, axis=0, keepdims=True)
    c_s[...] = (row_c == 0).astype(jnp.float32) * p0r
    ct_s[...] = p0 * (lane_c == 0).astype(jnp.float32)

    def body(i, _):
        dv = d_s[...]
        m = jnp.max(dv)
        li = jnp.min(jnp.where(dv == m, lane, _BIGI))
        sel = (lane == li).astype(jnp.float32)
        px = jnp.sum(posT * sel, axis=1, keepdims=True)
        pxr = jnp.concatenate([px[0:1, :], px[1:2, :], px[2:3, :]], axis=1)
        c_s[...] += (row_c == i).astype(jnp.float32) * pxr
        ct_s[...] += px * (lane_c == i).astype(jnp.float32)
        d_s[...] = jnp.minimum(dv, jnp.sum((posT - px) ** 2, axis=0,
                                           keepdims=True))
        return 0

    jax.lax.fori_loop(1, n_centers, body, 0)


def _ball_mlp_max(d2_s, acc_s, t, s_c, w2, b2, K, r_sq, n_centers, n_pts):
    lane = jax.lax.broadcasted_iota(jnp.int32, (n_centers, n_pts), 1)
    acc_s[...] = jnp.full(acc_s.shape, -_BIGF, dtype=jnp.float32)

    def body(k, _):
        d2v = d2_s[...]
        v, sel = _row_argmin_onehot(d2v, lane)
        d2_s[...] = jnp.where(sel, _BIGF, d2v)
        g = _dot(sel.astype(jnp.float32), t)
        h = _relu(_dot(_relu(g + s_c), w2) + b2)
        inball = v <= r_sq
        acc_s[...] = jnp.where(inball, jnp.maximum(acc_s[...], h), acc_s[...])
        return 0

    jax.lax.fori_loop(0, K, body, 0)


def _knn3_weights(d2, n_rows, n_src):
    lane = jax.lax.broadcasted_iota(jnp.int32, (n_rows, n_src), 1)
    wm = jnp.zeros((n_rows, n_src), dtype=jnp.float32)
    wsum = jnp.zeros((n_rows, 1), dtype=jnp.float32)
    for k in range(3):
        v, sel = _row_argmin_onehot(d2, lane)
        w = 1.0 / (v + 1e-8)
        wm = wm + w * sel.astype(jnp.float32)
        wsum = wsum + w
        if k < 2:
            d2 = jnp.where(sel, _BIGF, d2)
    return wm / wsum


def _stage_a(pos_ref, posT_ref, x_ref,
             w1a_ref, w1b_ref, b1_ref, w2_ref, b2_ref, wg_ref, bg_ref,
             x1_ref, c1_ref, c1t_ref,
             d_s, c_s, ct_s, d2_s, acc_s):
    pos = pos_ref[0]
    posT = posT_ref[0]
    xf = x_ref[0]

    _fps(posT, _N0, _N1, d_s, c_s, ct_s)
    c1 = c_s[...]

    d2_s[...] = _pairwise_d2(c1, posT)
    t = _dot(xf, w1a_ref[...]) + _dot(pos, w1b_ref[...])
    s_c = b1_ref[...] - _dot(c1, w1b_ref[...])
    _ball_mlp_max(d2_s, acc_s, t, s_c, w2_ref[...], b2_ref[...],
                  _K1, _R1SQ, _N1, _N0)

    x1_ref[0] = _relu(_dot(acc_s[...], wg_ref[...]) + bg_ref[...])
    c1_ref[0] = c1
    c1t_ref[0] = ct_s[...]


def _stage_b(x1_ref, c1_ref, c1t_ref,
             a2w1a_ref, a2w1b_ref, a2b1_ref, a2w2_ref, a2b2_ref,
             a2wg_ref, a2bg_ref,
             a3w1a_ref, a3w1b_ref, a3b1_ref, a3w2_ref, a3b2_ref,
             a3wg_ref, a3bg_ref,
             f1wa_ref, f1wb_ref, f1b1_ref, f1w2_ref, f1b2_ref,
             f2wa_ref, f2wb_ref, f2b_ref,
             xf2_ref,
             d_s, c_s, ct_s, d2_s, acc_s):
    x1 = x1_ref[0]
    c1 = c1_ref[0]
    c1t = c1t_ref[0]

    _fps(c1t, _N1, _N2, d_s, c_s, ct_s)
    c2 = c_s[...]
    c2t = ct_s[...]

    d2_s[...] = _pairwise_d2(c2, c1t)
    t = _dot(x1, a2w1a_ref[...]) + _dot(c1, a2w1b_ref[...])
    s_c = a2b1_ref[...] - _dot(c2, a2w1b_ref[...])
    _ball_mlp_max(d2_s, acc_s, t, s_c, a2w2_ref[...], a2b2_ref[...],
                  _K2, _R2SQ, _N2, _N1)
    x2 = _relu(_dot(acc_s[...], a2wg_ref[...]) + a2bg_ref[...])

    c3r = c2[0:1, :]
    h = _relu(_dot(x2, a3w1a_ref[...]) + _dot(c2 - c3r, a3w1b_ref[...])
              + a3b1_ref[...])
    h = _relu(_dot(h, a3w2_ref[...]) + a3b2_ref[...])
    hm = jnp.max(h, axis=0, keepdims=True)
    x3 = _relu(_dot(hm, a3wg_ref[...]) + a3bg_ref[...])

    r1 = _dot(x3, f1wa_ref[...])
    h = _relu(r1 + _dot(c2, f1wb_ref[...]) + f1b1_ref[...])
    xfp1 = _relu(_dot(h, f1w2_ref[...]) + f1b2_ref[...])

    d2 = _pairwise_d2(c1, c2t)
    wm = _knn3_weights(d2, _N1, _N2)
    interp = _dot(wm, xfp1)
    xf2_ref[0] = _relu(_dot(interp, f2wa_ref[...])
                       + _dot(c1, f2wb_ref[...]) + f2b_ref[...])


def _stage_c(pos_ref, xf2_ref, c1t_ref,
             w1a_ref, w1b_ref, b1_ref, w2_ref, b2_ref, w3_ref, b3_ref,
             w4_ref, b4_ref,
             out_ref):
    pos = pos_ref[0]
    xf2 = xf2_ref[0]
    c1t = c1t_ref[0]

    d2 = _pairwise_d2(pos, c1t)
    wm = _knn3_weights(d2, _N0, _N1)
    interp = _dot(wm, xf2)

    h = _relu(_dot(interp, w1a_ref[...]) + _dot(pos, w1b_ref[...]) + b1_ref[...])
    h = _relu(_dot(h, w2_ref[...]) + b2_ref[...])
    h = _relu(_dot(h, w3_ref[...]) + b3_ref[...])
    out_ref[0] = _dot(h, w4_ref[...]) + b4_ref[...]


def _full(shape):
    nd = len(shape)
    return pl.BlockSpec(shape, lambda i: (0,) * nd)


def _per_sample(shape):
    nd = len(shape)
    return pl.BlockSpec((1,) + shape, lambda i: (i,) + (0,) * nd)


def _cparams():
    return pltpu.CompilerParams(dimension_semantics=("arbitrary",))


def kernel(vertex_features, vertices, params):
    sa = params["sa"]
    fp = params["fp"]

    (a1w1, a1b1), (a1w2, a1b2) = sa[0]["local"]
    (a1wg, a1bg), = sa[0]["global"]
    (a2w1, a2b1), (a2w2, a2b2) = sa[1]["local"]
    (a2wg, a2bg), = sa[1]["global"]
    (a3w1, a3b1), (a3w2, a3b2) = sa[2]["local"]
    (a3wg, a3bg), = sa[2]["global"]
    (f1w1, f1b1), (f1w2, f1b2) = fp[0]
    (f2w1, f2b1), = fp[1]
    (f3w1, f3b1), (f3w2, f3b2), (f3w3, f3b3), (f3w4, f3b4) = fp[2]

    r2 = lambda b: b.reshape(1, -1)
    pos = vertices
    posT = vertices.transpose(0, 2, 1)
    xf = vertex_features

    a_weights = [a1w1[:3], a1w1[3:], r2(a1b1), a1w2, r2(a1b2), a1wg, r2(a1bg)]
    x1, c1, c1t = pl.pallas_call(
        _stage_a,
        grid=(_B,),
        in_specs=[_per_sample((_N0, 3)), _per_sample((3, _N0)),
                  _per_sample((_N0, 3))] + [_full(w.shape) for w in a_weights],
        out_specs=[_per_sample((_N1, 128)), _per_sample((_N1, 3)),
                   _per_sample((3, _N1))],
        out_shape=[jax.ShapeDtypeStruct((_B, _N1, 128), jnp.float32),
                   jax.ShapeDtypeStruct((_B, _N1, 3), jnp.float32),
                   jax.ShapeDtypeStruct((_B, 3, _N1), jnp.float32)],
        scratch_shapes=[pltpu.VMEM((1, _N0), jnp.float32),
                        pltpu.VMEM((_N1, 3), jnp.float32),
                        pltpu.VMEM((3, _N1), jnp.float32),
                        pltpu.VMEM((_N1, _N0), jnp.float32),
                        pltpu.VMEM((_N1, 64), jnp.float32)],
        compiler_params=_cparams(),
    )(pos, posT, xf, *a_weights)

    b_weights = [a2w1[:128], a2w1[128:], r2(a2b1), a2w2, r2(a2b2),
                 a2wg, r2(a2bg),
                 a3w1[:256], a3w1[256:], r2(a3b1), a3w2, r2(a3b2),
                 a3wg, r2(a3bg),
                 f1w1[:1024], f1w1[1024:], r2(f1b1), f1w2, r2(f1b2),
                 f2w1[:256], f2w1[256:], r2(f2b1)]
    xf2 = pl.pallas_call(
        _stage_b,
        grid=(_B,),
        in_specs=[_per_sample((_N1, 128)), _per_sample((_N1, 3)),
                  _per_sample((3, _N1))] + [_full(w.shape) for w in b_weights],
        out_specs=_per_sample((_N1, 128)),
        out_shape=jax.ShapeDtypeStruct((_B, _N1, 128), jnp.float32),
        scratch_shapes=[pltpu.VMEM((1, _N1), jnp.float32),
                        pltpu.VMEM((_N2, 3), jnp.float32),
                        pltpu.VMEM((3, _N2), jnp.float32),
                        pltpu.VMEM((_N2, _N1), jnp.float32),
                        pltpu.VMEM((_N2, 128), jnp.float32)],
        compiler_params=_cparams(),
    )(x1, c1, c1t, *b_weights)

    c_weights = [f3w1[:128], f3w1[128:], r2(f3b1), f3w2, r2(f3b2),
                 f3w3, r2(f3b3), f3w4, r2(f3b4)]
    out = pl.pallas_call(
        _stage_c,
        grid=(_B,),
        in_specs=[_per_sample((_N0, 3)), _per_sample((_N1, 128)),
                  _per_sample((3, _N1))] + [_full(w.shape) for w in c_weights],
        out_specs=_per_sample((_N0, 13)),
        out_shape=jax.ShapeDtypeStruct((_B, _N0, 13), jnp.float32),
        compiler_params=_cparams(),
    )(pos, xf2, c1t, *c_weights)

    return out


# speedup vs baseline: 5.9997x; 1.1783x over previous
import jax
import jax.numpy as jnp
from jax.experimental import pallas as pl
from jax.experimental.pallas import tpu as pltpu

_B = 16
_N0 = 4096
_N1 = 512
_N2 = 128
_K1 = 32
_K2 = 64
_R1SQ = 0.2 * 0.2
_R2SQ = 0.4 * 0.4
_BIGF = 1e30
_BIGI = 2 ** 30


def _relu(x):
    return jnp.maximum(x, 0.0)


def _dot(a, b):
    return jax.lax.dot_general(a, b, (((1,), (0,)), ((), ())),
                               preferred_element_type=jnp.float32)


def _row_argmin_onehot(d2, lane):
    v = jnp.min(d2, axis=1, keepdims=True)
    li = jnp.argmin(d2, axis=1)[:, None].astype(jnp.int32)
    sel = lane == li
    return v, sel


def _pairwise_d2(c, pT):
    d0 = (c[:, 0:1] - pT[0:1, :]) ** 2
    d1 = (c[:, 1:2] - pT[1:2, :]) ** 2
    d2 = (c[:, 2:3] - pT[2:3, :]) ** 2
    return (d0 + d1) + d2


def _fps(pos_ref, posT, n_pts, n_centers, d_s, c_s, ct_s):
    lane_c = jax.lax.broadcasted_iota(jnp.int32, (1, n_centers), 1)
    p0 = posT[:, 0:1]
    d_s[...] = jnp.sum((posT - p0) ** 2, axis=0, keepdims=True)
    c_s[0:1, :] = pos_ref[0, 0:1, :]
    ct_s[...] = p0 * (lane_c == 0).astype(jnp.float32)

    def body(i, _):
        dv = d_s[...]
        li = jnp.argmax(dv).astype(jnp.int32)
        pr = pos_ref[0, pl.ds(li, 1), :]
        c_s[pl.ds(i, 1), :] = pr
        px = jnp.concatenate([pr[:, 0:1], pr[:, 1:2], pr[:, 2:3]], axis=0)
        ct_s[...] += px * (lane_c == i).astype(jnp.float32)
        d_s[...] = jnp.minimum(dv, jnp.sum((posT - px) ** 2, axis=0,
                                           keepdims=True))
        return 0

    jax.lax.fori_loop(1, n_centers, body, 0)


def _ball_mlp_max(d2_s, acc_s, t, s_c, w2, b2, K, r_sq, n_centers, n_pts):
    lane = jax.lax.broadcasted_iota(jnp.int32, (n_centers, n_pts), 1)
    acc_s[...] = jnp.full(acc_s.shape, -_BIGF, dtype=jnp.float32)

    def body(k, _):
        d2v = d2_s[...]
        v, sel = _row_argmin_onehot(d2v, lane)
        d2_s[...] = jnp.where(sel, _BIGF, d2v)
        g = _dot(sel.astype(jnp.float32), t)
        h = _relu(_dot(_relu(g + s_c), w2) + b2)
        inball = v <= r_sq
        acc_s[...] = jnp.where(inball, jnp.maximum(acc_s[...], h), acc_s[...])
        return 0

    jax.lax.fori_loop(0, K, body, 0)


def _knn3_weights(d2, n_rows, n_src):
    lane = jax.lax.broadcasted_iota(jnp.int32, (n_rows, n_src), 1)
    wm = jnp.zeros((n_rows, n_src), dtype=jnp.float32)
    wsum = jnp.zeros((n_rows, 1), dtype=jnp.float32)
    for k in range(3):
        v, sel = _row_argmin_onehot(d2, lane)
        w = 1.0 / (v + 1e-8)
        wm = wm + w * sel.astype(jnp.float32)
        wsum = wsum + w
        if k < 2:
            d2 = jnp.where(sel, _BIGF, d2)
    return wm / wsum


def _stage_a(pos_ref, posT_ref, x_ref,
             w1a_ref, w1b_ref, b1_ref, w2_ref, b2_ref, wg_ref, bg_ref,
             x1_ref, c1_ref, c1t_ref,
             d_s, c_s, ct_s, d2_s, acc_s):
    pos = pos_ref[0]
    posT = posT_ref[0]
    xf = x_ref[0]

    _fps(pos_ref, posT, _N0, _N1, d_s, c_s, ct_s)
    c1 = c_s[...]

    d2_s[...] = _pairwise_d2(c1, posT)
    t = _dot(xf, w1a_ref[...]) + _dot(pos, w1b_ref[...])
    s_c = b1_ref[...] - _dot(c1, w1b_ref[...])
    _ball_mlp_max(d2_s, acc_s, t, s_c, w2_ref[...], b2_ref[...],
                  _K1, _R1SQ, _N1, _N0)

    x1_ref[0] = _relu(_dot(acc_s[...], wg_ref[...]) + bg_ref[...])
    c1_ref[0] = c1
    c1t_ref[0] = ct_s[...]


def _stage_b(x1_ref, c1_ref, c1t_ref,
             a2w1a_ref, a2w1b_ref, a2b1_ref, a2w2_ref, a2b2_ref,
             a2wg_ref, a2bg_ref,
             a3w1a_ref, a3w1b_ref, a3b1_ref, a3w2_ref, a3b2_ref,
             a3wg_ref, a3bg_ref,
             f1wa_ref, f1wb_ref, f1b1_ref, f1w2_ref, f1b2_ref,
             f2wa_ref, f2wb_ref, f2b_ref,
             xf2_ref,
             d_s, c_s, ct_s, d2_s, acc_s):
    x1 = x1_ref[0]
    c1 = c1_ref[0]
    c1t = c1t_ref[0]

    _fps(c1_ref, c1t, _N1, _N2, d_s, c_s, ct_s)
    c2 = c_s[...]
    c2t = ct_s[...]

    d2_s[...] = _pairwise_d2(c2, c1t)
    t = _dot(x1, a2w1a_ref[...]) + _dot(c1, a2w1b_ref[...])
    s_c = a2b1_ref[...] - _dot(c2, a2w1b_ref[...])
    _ball_mlp_max(d2_s, acc_s, t, s_c, a2w2_ref[...], a2b2_ref[...],
                  _K2, _R2SQ, _N2, _N1)
    x2 = _relu(_dot(acc_s[...], a2wg_ref[...]) + a2bg_ref[...])

    c3r = c2[0:1, :]
    h = _relu(_dot(x2, a3w1a_ref[...]) + _dot(c2 - c3r, a3w1b_ref[...])
              + a3b1_ref[...])
    h = _relu(_dot(h, a3w2_ref[...]) + a3b2_ref[...])
    hm = jnp.max(h, axis=0, keepdims=True)
    x3 = _relu(_dot(hm, a3wg_ref[...]) + a3bg_ref[...])

    r1 = _dot(x3, f1wa_ref[...])
    h = _relu(r1 + _dot(c2, f1wb_ref[...]) + f1b1_ref[...])
    xfp1 = _relu(_dot(h, f1w2_ref[...]) + f1b2_ref[...])

    d2 = _pairwise_d2(c1, c2t)
    wm = _knn3_weights(d2, _N1, _N2)
    interp = _dot(wm, xfp1)
    xf2_ref[0] = _relu(_dot(interp, f2wa_ref[...])
                       + _dot(c1, f2wb_ref[...]) + f2b_ref[...])


def _stage_c(pos_ref, xf2_ref, c1t_ref,
             w1a_ref, w1b_ref, b1_ref, w2_ref, b2_ref, w3_ref, b3_ref,
             w4_ref, b4_ref,
             out_ref):
    pos = pos_ref[0]
    xf2 = xf2_ref[0]
    c1t = c1t_ref[0]

    d2 = _pairwise_d2(pos, c1t)
    wm = _knn3_weights(d2, _N0, _N1)
    interp = _dot(wm, xf2)

    h = _relu(_dot(interp, w1a_ref[...]) + _dot(pos, w1b_ref[...]) + b1_ref[...])
    h = _relu(_dot(h, w2_ref[...]) + b2_ref[...])
    h = _relu(_dot(h, w3_ref[...]) + b3_ref[...])
    out_ref[0] = _dot(h, w4_ref[...]) + b4_ref[...]


def _full(shape):
    nd = len(shape)
    return pl.BlockSpec(shape, lambda i: (0,) * nd)


def _per_sample(shape):
    nd = len(shape)
    return pl.BlockSpec((1,) + shape, lambda i: (i,) + (0,) * nd)


def _cparams():
    return pltpu.CompilerParams(dimension_semantics=("arbitrary",))


def kernel(vertex_features, vertices, params):
    sa = params["sa"]
    fp = params["fp"]

    (a1w1, a1b1), (a1w2, a1b2) = sa[0]["local"]
    (a1wg, a1bg), = sa[0]["global"]
    (a2w1, a2b1), (a2w2, a2b2) = sa[1]["local"]
    (a2wg, a2bg), = sa[1]["global"]
    (a3w1, a3b1), (a3w2, a3b2) = sa[2]["local"]
    (a3wg, a3bg), = sa[2]["global"]
    (f1w1, f1b1), (f1w2, f1b2) = fp[0]
    (f2w1, f2b1), = fp[1]
    (f3w1, f3b1), (f3w2, f3b2), (f3w3, f3b3), (f3w4, f3b4) = fp[2]

    r2 = lambda b: b.reshape(1, -1)
    pos = vertices
    posT = vertices.transpose(0, 2, 1)
    xf = vertex_features

    a_weights = [a1w1[:3], a1w1[3:], r2(a1b1), a1w2, r2(a1b2), a1wg, r2(a1bg)]
    x1, c1, c1t = pl.pallas_call(
        _stage_a,
        grid=(_B,),
        in_specs=[_per_sample((_N0, 3)), _per_sample((3, _N0)),
                  _per_sample((_N0, 3))] + [_full(w.shape) for w in a_weights],
        out_specs=[_per_sample((_N1, 128)), _per_sample((_N1, 3)),
                   _per_sample((3, _N1))],
        out_shape=[jax.ShapeDtypeStruct((_B, _N1, 128), jnp.float32),
                   jax.ShapeDtypeStruct((_B, _N1, 3), jnp.float32),
                   jax.ShapeDtypeStruct((_B, 3, _N1), jnp.float32)],
        scratch_shapes=[pltpu.VMEM((1, _N0), jnp.float32),
                        pltpu.VMEM((_N1, 3), jnp.float32),
                        pltpu.VMEM((3, _N1), jnp.float32),
                        pltpu.VMEM((_N1, _N0), jnp.float32),
                        pltpu.VMEM((_N1, 64), jnp.float32)],
        compiler_params=_cparams(),
    )(pos, posT, xf, *a_weights)

    b_weights = [a2w1[:128], a2w1[128:], r2(a2b1), a2w2, r2(a2b2),
                 a2wg, r2(a2bg),
                 a3w1[:256], a3w1[256:], r2(a3b1), a3w2, r2(a3b2),
                 a3wg, r2(a3bg),
                 f1w1[:1024], f1w1[1024:], r2(f1b1), f1w2, r2(f1b2),
                 f2w1[:256], f2w1[256:], r2(f2b1)]
    xf2 = pl.pallas_call(
        _stage_b,
        grid=(_B,),
        in_specs=[_per_sample((_N1, 128)), _per_sample((_N1, 3)),
                  _per_sample((3, _N1))] + [_full(w.shape) for w in b_weights],
        out_specs=_per_sample((_N1, 128)),
        out_shape=jax.ShapeDtypeStruct((_B, _N1, 128), jnp.float32),
        scratch_shapes=[pltpu.VMEM((1, _N1), jnp.float32),
                        pltpu.VMEM((_N2, 3), jnp.float32),
                        pltpu.VMEM((3, _N2), jnp.float32),
                        pltpu.VMEM((_N2, _N1), jnp.float32),
                        pltpu.VMEM((_N2, 128), jnp.float32)],
        compiler_params=_cparams(),
    )(x1, c1, c1t, *b_weights)

    c_weights = [f3w1[:128], f3w1[128:], r2(f3b1), f3w2, r2(f3b2),
                 f3w3, r2(f3b3), f3w4, r2(f3b4)]
    out = pl.pallas_call(
        _stage_c,
        grid=(_B,),
        in_specs=[_per_sample((_N0, 3)), _per_sample((_N1, 128)),
                  _per_sample((3, _N1))] + [_full(w.shape) for w in c_weights],
        out_specs=_per_sample((_N0, 13)),
        out_shape=jax.ShapeDtypeStruct((_B, _N0, 13), jnp.float32),
        compiler_params=_cparams(),
    )(pos, xf2, c1t, *c_weights)

    return out


# speedup vs baseline: 6.0486x; 1.0082x over previous
import jax
import jax.numpy as jnp
from jax.experimental import pallas as pl
from jax.experimental.pallas import tpu as pltpu

_B = 16
_N0 = 4096
_N1 = 512
_N2 = 128
_K1 = 32
_K2 = 64
_R1SQ = 0.2 * 0.2
_R2SQ = 0.4 * 0.4
_BIGF = 1e30
_BIGI = 2 ** 30


def _relu(x):
    return jnp.maximum(x, 0.0)


def _dot(a, b):
    return jax.lax.dot_general(a, b, (((1,), (0,)), ((), ())),
                               preferred_element_type=jnp.float32)


def _row_argmin_onehot(d2, lane):
    v = jnp.min(d2, axis=1, keepdims=True)
    li = jnp.argmin(d2, axis=1)[:, None].astype(jnp.int32)
    sel = lane == li
    return v, sel


def _pairwise_d2(c, pT):
    d0 = (c[:, 0:1] - pT[0:1, :]) ** 2
    d1 = (c[:, 1:2] - pT[1:2, :]) ** 2
    d2 = (c[:, 2:3] - pT[2:3, :]) ** 2
    return (d0 + d1) + d2


def _fps(pos_ref, posT, n_pts, n_centers, d_s, c_s, ct_s):
    lane_c = jax.lax.broadcasted_iota(jnp.int32, (1, n_centers), 1)
    p0 = posT[:, 0:1]
    d_s[...] = jnp.sum((posT - p0) ** 2, axis=0, keepdims=True)
    c_s[0:1, :] = pos_ref[0, 0:1, :]
    ct_s[...] = p0 * (lane_c == 0).astype(jnp.float32)

    def body(i, _):
        dv = d_s[...]
        li = jnp.argmax(dv).astype(jnp.int32)
        pr = pos_ref[0, pl.ds(li, 1), :]
        c_s[pl.ds(i, 1), :] = pr
        px = jnp.concatenate([pr[:, 0:1], pr[:, 1:2], pr[:, 2:3]], axis=0)
        ct_s[...] += px * (lane_c == i).astype(jnp.float32)
        d_s[...] = jnp.minimum(dv, jnp.sum((posT - px) ** 2, axis=0,
                                           keepdims=True))
        return 0

    jax.lax.fori_loop(1, n_centers, body, 0)


def _ball_mlp_max(d2_s, acc_s, t_aug, c, s_c, w2, b2, K, r_sq,
                  n_centers, n_pts, fdim):
    lane = jax.lax.broadcasted_iota(jnp.int32, (n_centers, n_pts), 1)
    acc_s[...] = jnp.full(acc_s.shape, -_BIGF, dtype=jnp.float32)

    def body(k, _):
        d2v = d2_s[...]
        li = jnp.argmin(d2v, axis=1)[:, None].astype(jnp.int32)
        sel = lane == li
        d2_s[...] = jnp.where(sel, _BIGF, d2v)
        ga = _dot(sel.astype(jnp.float32), t_aug)
        g = ga[:, 0:fdim]
        gp = ga[:, fdim:fdim + 3]
        v = (((c[:, 0:1] - gp[:, 0:1]) ** 2
              + (c[:, 1:2] - gp[:, 1:2]) ** 2)
             + (c[:, 2:3] - gp[:, 2:3]) ** 2)
        h = _relu(_dot(_relu(g + s_c), w2) + b2)
        inball = v <= r_sq
        acc_s[...] = jnp.where(inball, jnp.maximum(acc_s[...], h), acc_s[...])
        return 0

    jax.lax.fori_loop(0, K, body, 0)


def _knn3_weights(d2, n_rows, n_src):
    lane = jax.lax.broadcasted_iota(jnp.int32, (n_rows, n_src), 1)
    wm = jnp.zeros((n_rows, n_src), dtype=jnp.float32)
    wsum = jnp.zeros((n_rows, 1), dtype=jnp.float32)
    for k in range(3):
        v, sel = _row_argmin_onehot(d2, lane)
        w = 1.0 / (v + 1e-8)
        wm = wm + w * sel.astype(jnp.float32)
        wsum = wsum + w
        if k < 2:
            d2 = jnp.where(sel, _BIGF, d2)
    return wm / wsum


def _stage_a(pos_ref, posT_ref, x_ref,
             w1a_ref, w1b_ref, b1_ref, w2_ref, b2_ref, wg_ref, bg_ref,
             x1_ref, c1_ref, c1t_ref,
             d_s, c_s, ct_s, d2_s, acc_s):
    pos = pos_ref[0]
    posT = posT_ref[0]
    xf = x_ref[0]

    _fps(pos_ref, posT, _N0, _N1, d_s, c_s, ct_s)
    c1 = c_s[...]

    d2_s[...] = _pairwise_d2(c1, posT)
    t = _dot(xf, w1a_ref[...]) + _dot(pos, w1b_ref[...])
    t_aug = jnp.concatenate([t, pos], axis=1)
    s_c = b1_ref[...] - _dot(c1, w1b_ref[...])
    _ball_mlp_max(d2_s, acc_s, t_aug, c1, s_c, w2_ref[...], b2_ref[...],
                  _K1, _R1SQ, _N1, _N0, 64)

    x1_ref[0] = _relu(_dot(acc_s[...], wg_ref[...]) + bg_ref[...])
    c1_ref[0] = c1
    c1t_ref[0] = ct_s[...]


def _stage_b(x1_ref, c1_ref, c1t_ref,
             a2w1a_ref, a2w1b_ref, a2b1_ref, a2w2_ref, a2b2_ref,
             a2wg_ref, a2bg_ref,
             a3w1a_ref, a3w1b_ref, a3b1_ref, a3w2_ref, a3b2_ref,
             a3wg_ref, a3bg_ref,
             f1wa_ref, f1wb_ref, f1b1_ref, f1w2_ref, f1b2_ref,
             f2wa_ref, f2wb_ref, f2b_ref,
             xf2_ref,
             d_s, c_s, ct_s, d2_s, acc_s):
    x1 = x1_ref[0]
    c1 = c1_ref[0]
    c1t = c1t_ref[0]

    _fps(c1_ref, c1t, _N1, _N2, d_s, c_s, ct_s)
    c2 = c_s[...]
    c2t = ct_s[...]

    d2_s[...] = _pairwise_d2(c2, c1t)
    t = _dot(x1, a2w1a_ref[...]) + _dot(c1, a2w1b_ref[...])
    t_aug = jnp.concatenate([t, c1], axis=1)
    s_c = a2b1_ref[...] - _dot(c2, a2w1b_ref[...])
    _ball_mlp_max(d2_s, acc_s, t_aug, c2, s_c, a2w2_ref[...], a2b2_ref[...],
                  _K2, _R2SQ, _N2, _N1, 128)
    x2 = _relu(_dot(acc_s[...], a2wg_ref[...]) + a2bg_ref[...])

    c3r = c2[0:1, :]
    h = _relu(_dot(x2, a3w1a_ref[...]) + _dot(c2 - c3r, a3w1b_ref[...])
              + a3b1_ref[...])
    h = _relu(_dot(h, a3w2_ref[...]) + a3b2_ref[...])
    hm = jnp.max(h, axis=0, keepdims=True)
    x3 = _relu(_dot(hm, a3wg_ref[...]) + a3bg_ref[...])

    r1 = _dot(x3, f1wa_ref[...])
    h = _relu(r1 + _dot(c2, f1wb_ref[...]) + f1b1_ref[...])
    xfp1 = _relu(_dot(h, f1w2_ref[...]) + f1b2_ref[...])

    d2 = _pairwise_d2(c1, c2t)
    wm = _knn3_weights(d2, _N1, _N2)
    interp = _dot(wm, xfp1)
    xf2_ref[0] = _relu(_dot(interp, f2wa_ref[...])
                       + _dot(c1, f2wb_ref[...]) + f2b_ref[...])


def _stage_c(pos_ref, xf2_ref, c1t_ref,
             w1a_ref, w1b_ref, b1_ref, w2_ref, b2_ref, w3_ref, b3_ref,
             w4_ref, b4_ref,
             out_ref):
    pos = pos_ref[0]
    xf2 = xf2_ref[0]
    c1t = c1t_ref[0]

    d2 = _pairwise_d2(pos, c1t)
    wm = _knn3_weights(d2, _N0, _N1)
    interp = _dot(wm, xf2)

    h = _relu(_dot(interp, w1a_ref[...]) + _dot(pos, w1b_ref[...]) + b1_ref[...])
    h = _relu(_dot(h, w2_ref[...]) + b2_ref[...])
    h = _relu(_dot(h, w3_ref[...]) + b3_ref[...])
    out_ref[0] = _dot(h, w4_ref[...]) + b4_ref[...]


def _full(shape):
    nd = len(shape)
    return pl.BlockSpec(shape, lambda i: (0,) * nd)


def _per_sample(shape):
    nd = len(shape)
    return pl.BlockSpec((1,) + shape, lambda i: (i,) + (0,) * nd)


def _cparams():
    return pltpu.CompilerParams(dimension_semantics=("parallel",))


def kernel(vertex_features, vertices, params):
    sa = params["sa"]
    fp = params["fp"]

    (a1w1, a1b1), (a1w2, a1b2) = sa[0]["local"]
    (a1wg, a1bg), = sa[0]["global"]
    (a2w1, a2b1), (a2w2, a2b2) = sa[1]["local"]
    (a2wg, a2bg), = sa[1]["global"]
    (a3w1, a3b1), (a3w2, a3b2) = sa[2]["local"]
    (a3wg, a3bg), = sa[2]["global"]
    (f1w1, f1b1), (f1w2, f1b2) = fp[0]
    (f2w1, f2b1), = fp[1]
    (f3w1, f3b1), (f3w2, f3b2), (f3w3, f3b3), (f3w4, f3b4) = fp[2]

    r2 = lambda b: b.reshape(1, -1)
    pos = vertices
    posT = vertices.transpose(0, 2, 1)
    xf = vertex_features

    a_weights = [a1w1[:3], a1w1[3:], r2(a1b1), a1w2, r2(a1b2), a1wg, r2(a1bg)]
    x1, c1, c1t = pl.pallas_call(
        _stage_a,
        grid=(_B,),
        in_specs=[_per_sample((_N0, 3)), _per_sample((3, _N0)),
                  _per_sample((_N0, 3))] + [_full(w.shape) for w in a_weights],
        out_specs=[_per_sample((_N1, 128)), _per_sample((_N1, 3)),
                   _per_sample((3, _N1))],
        out_shape=[jax.ShapeDtypeStruct((_B, _N1, 128), jnp.float32),
                   jax.ShapeDtypeStruct((_B, _N1, 3), jnp.float32),
                   jax.ShapeDtypeStruct((_B, 3, _N1), jnp.float32)],
        scratch_shapes=[pltpu.VMEM((1, _N0), jnp.float32),
                        pltpu.VMEM((_N1, 3), jnp.float32),
                        pltpu.VMEM((3, _N1), jnp.float32),
                        pltpu.VMEM((_N1, _N0), jnp.float32),
                        pltpu.VMEM((_N1, 64), jnp.float32)],
        compiler_params=_cparams(),
    )(pos, posT, xf, *a_weights)

    b_weights = [a2w1[:128], a2w1[128:], r2(a2b1), a2w2, r2(a2b2),
                 a2wg, r2(a2bg),
                 a3w1[:256], a3w1[256:], r2(a3b1), a3w2, r2(a3b2),
                 a3wg, r2(a3bg),
                 f1w1[:1024], f1w1[1024:], r2(f1b1), f1w2, r2(f1b2),
                 f2w1[:256], f2w1[256:], r2(f2b1)]
    xf2 = pl.pallas_call(
        _stage_b,
        grid=(_B,),
        in_specs=[_per_sample((_N1, 128)), _per_sample((_N1, 3)),
                  _per_sample((3, _N1))] + [_full(w.shape) for w in b_weights],
        out_specs=_per_sample((_N1, 128)),
        out_shape=jax.ShapeDtypeStruct((_B, _N1, 128), jnp.float32),
        scratch_shapes=[pltpu.VMEM((1, _N1), jnp.float32),
                        pltpu.VMEM((_N2, 3), jnp.float32),
                        pltpu.VMEM((3, _N2), jnp.float32),
                        pltpu.VMEM((_N2, _N1), jnp.float32),
                        pltpu.VMEM((_N2, 128), jnp.float32)],
        compiler_params=_cparams(),
    )(x1, c1, c1t, *b_weights)

    c_weights = [f3w1[:128], f3w1[128:], r2(f3b1), f3w2, r2(f3b2),
                 f3w3, r2(f3b3), f3w4, r2(f3b4)]
    out = pl.pallas_call(
        _stage_c,
        grid=(_B,),
        in_specs=[_per_sample((_N0, 3)), _per_sample((_N1, 128)),
                  _per_sample((3, _N1))] + [_full(w.shape) for w in c_weights],
        out_specs=_per_sample((_N0, 13)),
        out_shape=jax.ShapeDtypeStruct((_B, _N0, 13), jnp.float32),
        compiler_params=_cparams(),
    )(pos, xf2, c1t, *c_weights)

    return out


# speedup vs baseline: 13.6458x; 2.2560x over previous
import jax
import jax.numpy as jnp
from jax.experimental import pallas as pl
from jax.experimental.pallas import tpu as pltpu

_B = 16
_N0 = 4096
_N1 = 512
_N2 = 128
_K1 = 32
_K2 = 64
_R1SQ = 0.2 * 0.2
_R2SQ = 0.4 * 0.4
_BIGF = 1e30
_BIGI = 2 ** 30


def _relu(x):
    return jnp.maximum(x, 0.0)


def _dot(a, b):
    return jax.lax.dot_general(a, b, (((1,), (0,)), ((), ())),
                               preferred_element_type=jnp.float32)


def _row_argmin_onehot(d2, lane):
    v = jnp.min(d2, axis=1, keepdims=True)
    li = jnp.argmin(d2, axis=1)[:, None].astype(jnp.int32)
    sel = lane == li
    return v, sel


def _pairwise_d2(c, pT):
    d0 = (c[:, 0:1] - pT[0:1, :]) ** 2
    d1 = (c[:, 1:2] - pT[1:2, :]) ** 2
    d2 = (c[:, 2:3] - pT[2:3, :]) ** 2
    return (d0 + d1) + d2


def _fps_batched(posT3, n_pts, n_centers):
    lane = jax.lax.broadcasted_iota(jnp.int32, (_B, n_pts), 1)
    lane_c = jax.lax.broadcasted_iota(jnp.int32, (1, 1, n_centers), 2)
    p0 = posT3[:, :, 0:1]
    dsq = (posT3 - p0) ** 2
    d0 = (dsq[0] + dsq[1]) + dsq[2]
    ct0 = p0 * (lane_c == 0).astype(jnp.float32)

    def body(i, carry):
        d, ct = carry
        li = jnp.argmax(d, axis=1)[:, None].astype(jnp.int32)
        sel = (lane == li).astype(jnp.float32)
        px = jnp.sum(posT3 * sel[None], axis=2, keepdims=True)
        ct = ct + px * (lane_c == i).astype(jnp.float32)
        dsq = (posT3 - px) ** 2
        dn = (dsq[0] + dsq[1]) + dsq[2]
        return jnp.minimum(d, dn), ct

    _, ct = jax.lax.fori_loop(1, n_centers, body, (d0, ct0))
    return ct


def _fps_kernel(posT3_ref, c1t3_ref, c2t3_ref):
    c1t3 = _fps_batched(posT3_ref[...], _N0, _N1)
    c1t3_ref[...] = c1t3
    c2t3_ref[...] = _fps_batched(c1t3, _N1, _N2)


def _ball_mlp_max(d2_s, acc_s, t_aug, c, s_c, w2, b2, K, r_sq,
                  n_centers, n_pts, fdim):
    lane = jax.lax.broadcasted_iota(jnp.int32, (n_centers, n_pts), 1)
    acc_s[...] = jnp.full(acc_s.shape, -_BIGF, dtype=jnp.float32)

    def body(k, _):
        d2v = d2_s[...]
        li = jnp.argmin(d2v, axis=1)[:, None].astype(jnp.int32)
        sel = lane == li
        d2_s[...] = jnp.where(sel, _BIGF, d2v)
        ga = _dot(sel.astype(jnp.float32), t_aug)
        g = ga[:, 0:fdim]
        gp = ga[:, fdim:fdim + 3]
        v = (((c[:, 0:1] - gp[:, 0:1]) ** 2
              + (c[:, 1:2] - gp[:, 1:2]) ** 2)
             + (c[:, 2:3] - gp[:, 2:3]) ** 2)
        h = _relu(_dot(_relu(g + s_c), w2) + b2)
        inball = v <= r_sq
        acc_s[...] = jnp.where(inball, jnp.maximum(acc_s[...], h), acc_s[...])
        return 0

    jax.lax.fori_loop(0, K, body, 0)


def _knn3_weights(d2, n_rows, n_src):
    lane = jax.lax.broadcasted_iota(jnp.int32, (n_rows, n_src), 1)
    wm = jnp.zeros((n_rows, n_src), dtype=jnp.float32)
    wsum = jnp.zeros((n_rows, 1), dtype=jnp.float32)
    for k in range(3):
        v, sel = _row_argmin_onehot(d2, lane)
        w = 1.0 / (v + 1e-8)
        wm = wm + w * sel.astype(jnp.float32)
        wsum = wsum + w
        if k < 2:
            d2 = jnp.where(sel, _BIGF, d2)
    return wm / wsum


def _stage_a(pos_ref, posT_ref, x_ref, c1_ref,
             w1a_ref, w1b_ref, b1_ref, w2_ref, b2_ref, wg_ref, bg_ref,
             x1_ref,
             d2_s, acc_s):
    pos = pos_ref[0]
    posT = posT_ref[0]
    xf = x_ref[0]
    c1 = c1_ref[0]

    d2_s[...] = _pairwise_d2(c1, posT)
    t = _dot(xf, w1a_ref[...]) + _dot(pos, w1b_ref[...])
    t_aug = jnp.concatenate([t, pos], axis=1)
    s_c = b1_ref[...] - _dot(c1, w1b_ref[...])
    _ball_mlp_max(d2_s, acc_s, t_aug, c1, s_c, w2_ref[...], b2_ref[...],
                  _K1, _R1SQ, _N1, _N0, 64)

    x1_ref[0] = _relu(_dot(acc_s[...], wg_ref[...]) + bg_ref[...])


def _stage_b(x1_ref, c1_ref, c1t_ref, c2_ref, c2t_ref,
             a2w1a_ref, a2w1b_ref, a2b1_ref, a2w2_ref, a2b2_ref,
             a2wg_ref, a2bg_ref,
             a3w1a_ref, a3w1b_ref, a3b1_ref, a3w2_ref, a3b2_ref,
             a3wg_ref, a3bg_ref,
             f1wa_ref, f1wb_ref, f1b1_ref, f1w2_ref, f1b2_ref,
             f2wa_ref, f2wb_ref, f2b_ref,
             xf2_ref,
             d2_s, acc_s):
    x1 = x1_ref[0]
    c1 = c1_ref[0]
    c1t = c1t_ref[0]
    c2 = c2_ref[0]
    c2t = c2t_ref[0]

    d2_s[...] = _pairwise_d2(c2, c1t)
    t = _dot(x1, a2w1a_ref[...]) + _dot(c1, a2w1b_ref[...])
    t_aug = jnp.concatenate([t, c1], axis=1)
    s_c = a2b1_ref[...] - _dot(c2, a2w1b_ref[...])
    _ball_mlp_max(d2_s, acc_s, t_aug, c2, s_c, a2w2_ref[...], a2b2_ref[...],
                  _K2, _R2SQ, _N2, _N1, 128)
    x2 = _relu(_dot(acc_s[...], a2wg_ref[...]) + a2bg_ref[...])

    c3r = c2[0:1, :]
    h = _relu(_dot(x2, a3w1a_ref[...]) + _dot(c2 - c3r, a3w1b_ref[...])
              + a3b1_ref[...])
    h = _relu(_dot(h, a3w2_ref[...]) + a3b2_ref[...])
    hm = jnp.max(h, axis=0, keepdims=True)
    x3 = _relu(_dot(hm, a3wg_ref[...]) + a3bg_ref[...])

    r1 = _dot(x3, f1wa_ref[...])
    h = _relu(r1 + _dot(c2, f1wb_ref[...]) + f1b1_ref[...])
    xfp1 = _relu(_dot(h, f1w2_ref[...]) + f1b2_ref[...])

    d2 = _pairwise_d2(c1, c2t)
    wm = _knn3_weights(d2, _N1, _N2)
    interp = _dot(wm, xfp1)
    xf2_ref[0] = _relu(_dot(interp, f2wa_ref[...])
                       + _dot(c1, f2wb_ref[...]) + f2b_ref[...])


def _stage_c(pos_ref, xf2_ref, c1t_ref,
             w1a_ref, w1b_ref, b1_ref, w2_ref, b2_ref, w3_ref, b3_ref,
             w4_ref, b4_ref,
             out_ref):
    pos = pos_ref[0]
    xf2 = xf2_ref[0]
    c1t = c1t_ref[0]

    d2 = _pairwise_d2(pos, c1t)
    wm = _knn3_weights(d2, _N0, _N1)
    interp = _dot(wm, xf2)

    h = _relu(_dot(interp, w1a_ref[...]) + _dot(pos, w1b_ref[...]) + b1_ref[...])
    h = _relu(_dot(h, w2_ref[...]) + b2_ref[...])
    h = _relu(_dot(h, w3_ref[...]) + b3_ref[...])
    out_ref[0] = _dot(h, w4_ref[...]) + b4_ref[...]


def _full(shape):
    nd = len(shape)
    return pl.BlockSpec(shape, lambda i: (0,) * nd)


def _per_sample(shape):
    nd = len(shape)
    return pl.BlockSpec((1,) + shape, lambda i: (i,) + (0,) * nd)


def _cparams():
    return pltpu.CompilerParams(dimension_semantics=("parallel",))


def kernel(vertex_features, vertices, params):
    sa = params["sa"]
    fp = params["fp"]

    (a1w1, a1b1), (a1w2, a1b2) = sa[0]["local"]
    (a1wg, a1bg), = sa[0]["global"]
    (a2w1, a2b1), (a2w2, a2b2) = sa[1]["local"]
    (a2wg, a2bg), = sa[1]["global"]
    (a3w1, a3b1), (a3w2, a3b2) = sa[2]["local"]
    (a3wg, a3bg), = sa[2]["global"]
    (f1w1, f1b1), (f1w2, f1b2) = fp[0]
    (f2w1, f2b1), = fp[1]
    (f3w1, f3b1), (f3w2, f3b2), (f3w3, f3b3), (f3w4, f3b4) = fp[2]

    r2 = lambda b: b.reshape(1, -1)
    pos = vertices
    posT = vertices.transpose(0, 2, 1)
    posT3 = vertices.transpose(2, 0, 1)
    xf = vertex_features

    c1t3, c2t3 = pl.pallas_call(
        _fps_kernel,
        out_shape=[jax.ShapeDtypeStruct((3, _B, _N1), jnp.float32),
                   jax.ShapeDtypeStruct((3, _B, _N2), jnp.float32)],
    )(posT3)
    c1 = c1t3.transpose(1, 2, 0)
    c1t = c1t3.transpose(1, 0, 2)
    c2 = c2t3.transpose(1, 2, 0)
    c2t = c2t3.transpose(1, 0, 2)

    a_weights = [a1w1[:3], a1w1[3:], r2(a1b1), a1w2, r2(a1b2), a1wg, r2(a1bg)]
    x1 = pl.pallas_call(
        _stage_a,
        grid=(_B,),
        in_specs=[_per_sample((_N0, 3)), _per_sample((3, _N0)),
                  _per_sample((_N0, 3)), _per_sample((_N1, 3))]
                 + [_full(w.shape) for w in a_weights],
        out_specs=_per_sample((_N1, 128)),
        out_shape=jax.ShapeDtypeStruct((_B, _N1, 128), jnp.float32),
        scratch_shapes=[pltpu.VMEM((_N1, _N0), jnp.float32),
                        pltpu.VMEM((_N1, 64), jnp.float32)],
        compiler_params=_cparams(),
    )(pos, posT, xf, c1, *a_weights)

    b_weights = [a2w1[:128], a2w1[128:], r2(a2b1), a2w2, r2(a2b2),
                 a2wg, r2(a2bg),
                 a3w1[:256], a3w1[256:], r2(a3b1), a3w2, r2(a3b2),
                 a3wg, r2(a3bg),
                 f1w1[:1024], f1w1[1024:], r2(f1b1), f1w2, r2(f1b2),
                 f2w1[:256], f2w1[256:], r2(f2b1)]
    xf2 = pl.pallas_call(
        _stage_b,
        grid=(_B,),
        in_specs=[_per_sample((_N1, 128)), _per_sample((_N1, 3)),
                  _per_sample((3, _N1)), _per_sample((_N2, 3)),
                  _per_sample((3, _N2))] + [_full(w.shape) for w in b_weights],
        out_specs=_per_sample((_N1, 128)),
        out_shape=jax.ShapeDtypeStruct((_B, _N1, 128), jnp.float32),
        scratch_shapes=[pltpu.VMEM((_N2, _N1), jnp.float32),
                        pltpu.VMEM((_N2, 128), jnp.float32)],
        compiler_params=_cparams(),
    )(x1, c1, c1t, c2, c2t, *b_weights)

    c_weights = [f3w1[:128], f3w1[128:], r2(f3b1), f3w2, r2(f3b2),
                 f3w3, r2(f3b3), f3w4, r2(f3b4)]
    out = pl.pallas_call(
        _stage_c,
        grid=(_B,),
        in_specs=[_per_sample((_N0, 3)), _per_sample((_N1, 128)),
                  _per_sample((3, _N1))] + [_full(w.shape) for w in c_weights],
        out_specs=_per_sample((_N0, 13)),
        out_shape=jax.ShapeDtypeStruct((_B, _N0, 13), jnp.float32),
        compiler_params=_cparams(),
    )(pos, xf2, c1t, *c_weights)

    return out
